# Initial kernel scaffold; baseline (speedup 1.0000x reference)
import jax
import jax.numpy as jnp
from jax.experimental import pallas as pl


def kernel(q, p, n, emb, W, b):
    raise NotImplementedError("write your pallas kernel here")


# speedup vs baseline: 1.1687x; 1.1687x over previous
import functools

import jax
import jax.numpy as jnp
from jax import lax
from jax.experimental import pallas as pl
from jax.experimental.pallas import tpu as pltpu
from jax.experimental.pallas import tpu_sc as plsc

VOCAB = 100000
D = 300
DP = 304
HIST = 50
GL = 2 * HIST + 4
B = 4096
BT = 3 * B
NC = 2
NS = 16
NW = NC * NS
ROWS_PER_W = BT // NW
GROUPS_PER_W = ROWS_PER_W // 2
NGRP = BT // 2
GROUP = 32
LANES = 16
NCHUNK = DP // LANES

_mesh = plsc.VectorSubcoreMesh(core_axis_name="c", subcore_axis_name="s")


@functools.partial(
    pl.kernel,
    mesh=_mesh,
    out_type=jax.ShapeDtypeStruct((BT, DP), jnp.float32),
    scratch_types=[
        pltpu.VMEM((GROUPS_PER_W, GL), jnp.int32),
        pltpu.VMEM((GL, DP), jnp.float32),
        pltpu.VMEM((GL, DP), jnp.float32),
        pltpu.VMEM((GROUP, DP), jnp.float32),
        pltpu.SemaphoreType.DMA,
        pltpu.SemaphoreType.DMA,
    ],
    compiler_params=pltpu.CompilerParams(use_tc_tiling_on_sc=False),
)
def _pool_kernel(idx_hbm, emb_hbm, out_hbm, idx_v, rows0, rows1, outb, sem0, sem1):
    wid = lax.axis_index("s") * NC + lax.axis_index("c")
    base = pl.multiple_of(wid * ROWS_PER_W, ROWS_PER_W)
    gbase = pl.multiple_of(wid * GROUPS_PER_W, GROUPS_PER_W)
    pltpu.sync_copy(idx_hbm.at[pl.ds(gbase, GROUPS_PER_W)], idx_v)

    bufs = (rows0, rows1)
    sems = (sem0, sem1)

    pltpu.async_copy(emb_hbm.at[idx_v.at[0]], rows0, sem0)
    pltpu.async_copy(emb_hbm.at[idx_v.at[1]], rows1, sem1)

    scale = jnp.float32(1.0 / HIST)

    def pair_body(pair, _):
        for bi in range(2):
            g = pair * 2 + bi
            buf = bufs[bi]
            sem = sems[bi]
            pltpu.make_async_copy(emb_hbm.at[idx_v.at[g]], buf, sem).wait()

            def acc_body(r, accs):
                a = tuple(
                    acc + buf[r, pl.ds(j * LANES, LANES)]
                    for j, acc in enumerate(accs[:NCHUNK])
                )
                bacc = tuple(
                    acc + buf[HIST + r, pl.ds(j * LANES, LANES)]
                    for j, acc in enumerate(accs[NCHUNK:])
                )
                return a + bacc

            init = tuple(
                jnp.zeros((LANES,), jnp.float32) for _ in range(2 * NCHUNK)
            )
            accs = lax.fori_loop(0, HIST, acc_body, init)

            @pl.when(g + 2 < GROUPS_PER_W)
            def _():
                pltpu.async_copy(emb_hbm.at[idx_v.at[g + 2]], buf, sem)

            slot = lax.rem(2 * g, GROUP)
            for j in range(NCHUNK):
                outb[slot, pl.ds(j * LANES, LANES)] = accs[j] * scale
            for j in range(NCHUNK):
                outb[slot + 1, pl.ds(j * LANES, LANES)] = accs[NCHUNK + j] * scale

            @pl.when(slot == GROUP - 2)
            def _():
                flush_base = pl.multiple_of(base + 2 * g - (GROUP - 2), GROUP)
                pltpu.sync_copy(outb, out_hbm.at[pl.ds(flush_base, GROUP)])
        return 0

    lax.fori_loop(0, GROUPS_PER_W // 2, pair_body, 0)


def _proj_body(x_ref, w_ref, b_ref, o_ref):
    o_ref[...] = (
        lax.dot_general(
            x_ref[...],
            w_ref[...],
            (((1,), (1,)), ((), ())),
            preferred_element_type=jnp.float32,
        )
        + b_ref[...]
    )


_BLK = 1024


def _proj(pooled, W_pad, b2):
    return pl.pallas_call(
        _proj_body,
        grid=(BT // _BLK,),
        in_specs=[
            pl.BlockSpec((_BLK, DP), lambda i: (i, 0)),
            pl.BlockSpec((D, DP), lambda i: (0, 0)),
            pl.BlockSpec((1, D), lambda i: (0, 0)),
        ],
        out_specs=pl.BlockSpec((_BLK, D), lambda i: (i, 0)),
        out_shape=jax.ShapeDtypeStruct((BT, D), jnp.float32),
    )(pooled, W_pad, b2)


@jax.jit
def kernel(q, p, n, emb, W, b):
    idx_all = jnp.concatenate(
        [q.astype(jnp.int32), p.astype(jnp.int32), n.astype(jnp.int32)], axis=0
    )
    idx_pack = jnp.pad(idx_all.reshape(NGRP, 2 * HIST), ((0, 0), (0, 4)))
    emb_pad = jnp.pad(emb, ((0, 0), (0, DP - D)))
    W_pad = jnp.pad(W, ((0, 0), (0, DP - D)))
    pooled = _pool_kernel(idx_pack, emb_pad)
    out = _proj(pooled, W_pad, b.reshape(1, D))
    return (out[:B], out[B : 2 * B], out[2 * B :])
